# Initial kernel scaffold; baseline (speedup 1.0000x reference)
import jax
import jax.numpy as jnp
from jax.experimental import pallas as pl


def kernel(x, edge_index, num_graphs, W, b_conv, gamma, beta, fc_W, fc_b):
    raise NotImplementedError("write your pallas kernel here")


# speedup vs baseline: 6.6570x; 6.6570x over previous
import functools

import jax
import jax.numpy as jnp
from jax import lax
from jax.experimental import pallas as pl
from jax.experimental.pallas import tpu as pltpu
from jax.experimental.pallas import tpu_sc as plsc

NC, NS, LANES = 2, 16, 16


def _deg_kernel(dst, n_nodes):
    e = dst.shape[0]
    nw = NC * NS
    ept = e // nw
    mesh = plsc.VectorSubcoreMesh(core_axis_name="c", subcore_axis_name="s")

    @functools.partial(
        pl.kernel,
        out_type=jax.ShapeDtypeStruct((nw, n_nodes // LANES, LANES),
                                      jnp.float32),
        mesh=mesh,
        scratch_types=[
            pltpu.VMEM((n_nodes // LANES, LANES), jnp.float32),
            pltpu.VMEM((ept,), jnp.int32),
        ],
        compiler_params=pltpu.CompilerParams(needs_layout_passes=False),
    )
    def k(dst_hbm, out_hbm, deg_l, dst_v):
        c = lax.axis_index("c")
        s = lax.axis_index("s")
        wid = c * NS + s

        def zero(i, _):
            deg_l[i, :] = jnp.zeros((LANES,), jnp.float32)
            return 0

        lax.fori_loop(0, n_nodes // LANES, zero, 0)

        pltpu.sync_copy(dst_hbm.at[pl.ds(wid * ept, ept)], dst_v)
        ones = jnp.ones((LANES,), jnp.float32)

        def acc(g, _):
            idx = dst_v[pl.ds(g * LANES, LANES)]
            plsc.addupdate_scatter(
                deg_l, [idx >> 4, idx & (LANES - 1)], ones)
            return 0

        lax.fori_loop(0, ept // LANES, acc, 0)
        pltpu.sync_copy(deg_l, out_hbm.at[wid])

    return k(dst)


def _matmul_kernel(x, w, degp):
    n, d_in = x.shape
    dh = w.shape[1]
    half = dh // 2
    blk = 256
    nw = degp.shape[0]

    def body(x_ref, w_ref, degp_ref, y0_ref, y1_ref):
        deg = jnp.sum(degp_ref[...], axis=0) + 1.0
        dinv = lax.rsqrt(deg)
        y = jnp.dot(x_ref[...], w_ref[...], preferred_element_type=jnp.float32)
        y = y * dinv[:, None]
        y0_ref[...] = y[:, :half]
        y1_ref[...] = y[:, half:]

    return pl.pallas_call(
        body,
        grid=(n // blk,),
        in_specs=[
            pl.BlockSpec((blk, d_in), lambda i: (i, 0)),
            pl.BlockSpec((d_in, dh), lambda i: (0, 0)),
            pl.BlockSpec((nw, blk), lambda i: (0, i)),
        ],
        out_specs=[
            pl.BlockSpec((blk, half), lambda i: (i, 0)),
            pl.BlockSpec((blk, half), lambda i: (i, 0)),
        ],
        out_shape=[
            jax.ShapeDtypeStruct((n, half), jnp.float32),
            jax.ShapeDtypeStruct((n, half), jnp.float32),
        ],
    )(x, w, degp)


def _scatter_kernel(y0, y1, src2, dst2, n_nodes):
    nrows = src2.shape[0]
    rpt = nrows // NS
    stripe = n_nodes // NS
    zrows = 64
    mesh = plsc.VectorSubcoreMesh(core_axis_name="c", subcore_axis_name="s")

    @functools.partial(
        pl.kernel,
        out_type=(
            jax.ShapeDtypeStruct((n_nodes, 128), jnp.float32),
            jax.ShapeDtypeStruct((n_nodes, 128), jnp.float32),
        ),
        mesh=mesh,
        scratch_types=[
            pltpu.VMEM((rpt, 128), jnp.int32),
            pltpu.VMEM((rpt, 128), jnp.int32),
            pltpu.VMEM((128, 128), jnp.float32),
            pltpu.VMEM((zrows, 128), jnp.float32),
            pltpu.VMEM_SHARED((n_nodes, 128), jnp.float32),
        ],
    )
    def k(y0_hbm, y1_hbm, src_hbm, dst_hbm, a0_hbm, a1_hbm,
          src_v, dst_v, buf, zbuf, acc_sh):
        c = lax.axis_index("c")
        s = lax.axis_index("s")

        pltpu.sync_copy(src_hbm.at[pl.ds(s * rpt, rpt)], src_v)
        pltpu.sync_copy(dst_hbm.at[pl.ds(s * rpt, rpt)], dst_v)

        def zv(i, _):
            zbuf[i // 8, pl.ds((i % 8) * LANES, LANES)] = jnp.zeros(
                (LANES,), jnp.float32)
            return 0

        lax.fori_loop(0, zrows * 8, zv, 0)

        def zc(j, _):
            pltpu.sync_copy(zbuf, acc_sh.at[pl.ds(s * stripe + j * zrows, zrows)])
            return 0

        lax.fori_loop(0, stripe // zrows, zc, 0)
        plsc.subcore_barrier()

        def do_edges(y_hbm):
            def body(j, _):
                pltpu.sync_copy(y_hbm.at[src_v.at[j]], buf)
                pltpu.sync_copy(buf, acc_sh.at[dst_v.at[j]], add=True)
                return 0

            lax.fori_loop(0, rpt, body, 0)

        @pl.when(c == 0)
        def _():
            do_edges(y0_hbm)

        @pl.when(c == 1)
        def _():
            do_edges(y1_hbm)

        plsc.subcore_barrier()

        def wout(a_hbm):
            pltpu.sync_copy(acc_sh.at[pl.ds(s * stripe, stripe)],
                            a_hbm.at[pl.ds(s * stripe, stripe)])

        @pl.when(c == 0)
        def _():
            wout(a0_hbm)

        @pl.when(c == 1)
        def _():
            wout(a1_hbm)

    return k(y0, y1, src2, dst2)


def _bn_stats_kernel(y0, y1, acc0, acc1, degp, bc2):
    n, half = y0.shape
    dh = 2 * half
    blk = 512
    nw = degp.shape[0]

    def body(y0_ref, y1_ref, a0_ref, a1_ref, degp_ref, bc_ref, h_ref, st_ref):
        i = pl.program_id(0)
        deg = jnp.sum(degp_ref[...], axis=0) + 1.0
        dinv = lax.rsqrt(deg)[:, None]
        h0 = jnp.maximum(dinv * (y0_ref[...] + a0_ref[...]) + bc_ref[0:1, :], 0.0)
        h1 = jnp.maximum(dinv * (y1_ref[...] + a1_ref[...]) + bc_ref[1:2, :], 0.0)
        h_ref[:, 0:half] = h0
        h_ref[:, half:dh] = h1

        @pl.when(i == 0)
        def _():
            st_ref[...] = jnp.zeros_like(st_ref)

        row_s = jnp.concatenate([jnp.sum(h0, axis=0), jnp.sum(h1, axis=0)])
        row_q = jnp.concatenate(
            [jnp.sum(h0 * h0, axis=0), jnp.sum(h1 * h1, axis=0)])
        st_ref[0:1, :] += row_s[None, :]
        st_ref[1:2, :] += row_q[None, :]

    return pl.pallas_call(
        body,
        grid=(n // blk,),
        in_specs=[
            pl.BlockSpec((blk, half), lambda i: (i, 0)),
            pl.BlockSpec((blk, half), lambda i: (i, 0)),
            pl.BlockSpec((blk, half), lambda i: (i, 0)),
            pl.BlockSpec((blk, half), lambda i: (i, 0)),
            pl.BlockSpec((nw, blk), lambda i: (0, i)),
            pl.BlockSpec((2, half), lambda i: (0, 0)),
        ],
        out_specs=[
            pl.BlockSpec((blk, dh), lambda i: (i, 0)),
            pl.BlockSpec((8, dh), lambda i: (0, 0)),
        ],
        out_shape=[
            jax.ShapeDtypeStruct((n, dh), jnp.float32),
            jax.ShapeDtypeStruct((8, dh), jnp.float32),
        ],
    )(y0, y1, acc0, acc1, degp, bc2)


def _fc_kernel(hg, st, gamma1, beta1, fc_w, fcb1, n_total):
    g, d4 = hg.shape
    dh = d4 // 4
    ncls = fc_w.shape[0]
    blk = 256

    def body(hg_ref, st_ref, ga_ref, be_ref, fw_ref, fb_ref, out_ref):
        inv_n = 1.0 / float(n_total)
        mean = st_ref[0:1, :] * inv_n
        ex2 = st_ref[1:2, :] * inv_n
        var = ex2 - mean * mean
        rstd = lax.rsqrt(var + 1e-5)
        scale = ga_ref[0:1, :] * rstd
        off = be_ref[0:1, :] - mean * scale
        scale4 = jnp.concatenate([scale] * 4, axis=1)
        off4 = jnp.concatenate([off] * 4, axis=1)
        hgn = hg_ref[...] * scale4 + off4
        logits = lax.dot_general(
            hgn, fw_ref[...], (((1,), (1,)), ((), ())),
            preferred_element_type=jnp.float32) + fb_ref[0:1, :]
        m = jnp.max(logits, axis=1, keepdims=True)
        lse = m + jnp.log(jnp.sum(jnp.exp(logits - m), axis=1, keepdims=True))
        ls = logits - lse
        out_ref[...] = jnp.broadcast_to(ls[:, None, :], (blk, 4, ncls))

    return pl.pallas_call(
        body,
        grid=(g // blk,),
        in_specs=[
            pl.BlockSpec((blk, d4), lambda i: (i, 0)),
            pl.BlockSpec((8, dh), lambda i: (0, 0)),
            pl.BlockSpec((1, dh), lambda i: (0, 0)),
            pl.BlockSpec((1, dh), lambda i: (0, 0)),
            pl.BlockSpec((ncls, d4), lambda i: (0, 0)),
            pl.BlockSpec((1, ncls), lambda i: (0, 0)),
        ],
        out_specs=pl.BlockSpec((blk, 4, ncls), lambda i: (i, 0, 0)),
        out_shape=jax.ShapeDtypeStruct((g, 4, ncls), jnp.float32),
    )(hg, st, gamma1, beta1, fc_w, fcb1)


def kernel(x, edge_index, num_graphs, W, b_conv, gamma, beta, fc_W, fc_b):
    del num_graphs
    n, _ = x.shape
    dh = W.shape[1]
    src = edge_index[0]
    dst = edge_index[1]
    e = src.shape[0]

    degp = _deg_kernel(dst, n).reshape(NC * NS, n)
    y0, y1 = _matmul_kernel(x, W, degp)
    src2 = src.reshape(e // 128, 128)
    dst2 = dst.reshape(e // 128, 128)
    acc0, acc1 = _scatter_kernel(y0, y1, src2, dst2, n)
    bc2 = b_conv.reshape(2, dh // 2)
    h, st = _bn_stats_kernel(y0, y1, acc0, acc1, degp, bc2)
    hg = h.reshape(n // 4, 4 * dh)
    out3 = _fc_kernel(hg, st, gamma.reshape(1, dh), beta.reshape(1, dh),
                      fc_W, fc_b.reshape(1, -1), n)
    return out3.reshape(n, fc_W.shape[0])


# speedup vs baseline: 7.7407x; 1.1628x over previous
import functools

import jax
import jax.numpy as jnp
from jax import lax
from jax.experimental import pallas as pl
from jax.experimental.pallas import tpu as pltpu
from jax.experimental.pallas import tpu_sc as plsc

NC, NS, LANES = 2, 16, 16


def _deg_kernel(dst, n_nodes):
    e = dst.shape[0]
    nw = NC * NS
    ept = e // nw
    mesh = plsc.VectorSubcoreMesh(core_axis_name="c", subcore_axis_name="s")

    @functools.partial(
        pl.kernel,
        out_type=jax.ShapeDtypeStruct((nw, n_nodes), jnp.float32),
        mesh=mesh,
        scratch_types=[
            pltpu.VMEM((n_nodes,), jnp.float32),
            pltpu.VMEM((ept,), jnp.int32),
        ],
        compiler_params=pltpu.CompilerParams(needs_layout_passes=False),
    )
    def k(dst_hbm, out_hbm, deg_l, dst_v):
        c = lax.axis_index("c")
        s = lax.axis_index("s")
        wid = c * NS + s

        def zero(i, _):
            deg_l[pl.ds(i * LANES, LANES)] = jnp.zeros((LANES,), jnp.float32)
            return 0

        lax.fori_loop(0, n_nodes // LANES, zero, 0)

        pltpu.sync_copy(dst_hbm.at[pl.ds(wid * ept, ept)], dst_v)
        ones = jnp.ones((LANES,), jnp.float32)

        def acc(g, _):
            idx = dst_v[pl.ds(g * LANES, LANES)]
            plsc.addupdate_scatter(deg_l, [idx], ones)
            return 0

        lax.fori_loop(0, ept // LANES, acc, 0)
        pltpu.sync_copy(deg_l, out_hbm.at[wid])

    return k(dst)


def _matmul_kernel(x, w, degp):
    n, d_in = x.shape
    dh = w.shape[1]
    half = dh // 2
    blk = 256
    nw = degp.shape[0]

    def body(x_ref, w_ref, degp_ref, y0_ref, y1_ref):
        deg = jnp.sum(degp_ref[...], axis=0) + 1.0
        dinv = lax.rsqrt(deg)
        y = jnp.dot(x_ref[...], w_ref[...], preferred_element_type=jnp.float32)
        y = y * dinv[:, None]
        y0_ref[...] = y[:, :half]
        y1_ref[...] = y[:, half:]

    return pl.pallas_call(
        body,
        grid=(n // blk,),
        in_specs=[
            pl.BlockSpec((blk, d_in), lambda i: (i, 0)),
            pl.BlockSpec((d_in, dh), lambda i: (0, 0)),
            pl.BlockSpec((nw, blk), lambda i: (0, i)),
        ],
        out_specs=[
            pl.BlockSpec((blk, half), lambda i: (i, 0)),
            pl.BlockSpec((blk, half), lambda i: (i, 0)),
        ],
        out_shape=[
            jax.ShapeDtypeStruct((n, half), jnp.float32),
            jax.ShapeDtypeStruct((n, half), jnp.float32),
        ],
    )(x, w, degp)


def _scatter_kernel(y0, y1, src2, dst2, n_nodes):
    nrows = src2.shape[0]
    rpt = nrows // NS
    stripe = n_nodes // NS
    zrows = 64
    mesh = plsc.VectorSubcoreMesh(core_axis_name="c", subcore_axis_name="s")

    @functools.partial(
        pl.kernel,
        out_type=(
            jax.ShapeDtypeStruct((n_nodes, 128), jnp.float32),
            jax.ShapeDtypeStruct((n_nodes, 128), jnp.float32),
        ),
        mesh=mesh,
        scratch_types=[
            pltpu.VMEM((rpt, 128), jnp.int32),
            pltpu.VMEM((rpt, 128), jnp.int32),
            pltpu.VMEM((128, 128), jnp.float32),
            pltpu.VMEM((zrows, 128), jnp.float32),
            pltpu.VMEM_SHARED((n_nodes, 128), jnp.float32),
        ],
    )
    def k(y0_hbm, y1_hbm, src_hbm, dst_hbm, a0_hbm, a1_hbm,
          src_v, dst_v, buf, zbuf, acc_sh):
        c = lax.axis_index("c")
        s = lax.axis_index("s")

        pltpu.sync_copy(src_hbm.at[pl.ds(s * rpt, rpt)], src_v)
        pltpu.sync_copy(dst_hbm.at[pl.ds(s * rpt, rpt)], dst_v)

        def zv(i, _):
            zbuf[i // 8, pl.ds((i % 8) * LANES, LANES)] = jnp.zeros(
                (LANES,), jnp.float32)
            return 0

        lax.fori_loop(0, zrows * 8, zv, 0)

        def zc(j, _):
            pltpu.sync_copy(zbuf, acc_sh.at[pl.ds(s * stripe + j * zrows, zrows)])
            return 0

        lax.fori_loop(0, stripe // zrows, zc, 0)
        plsc.subcore_barrier()

        def do_edges(y_hbm):
            def body(j, _):
                pltpu.sync_copy(y_hbm.at[src_v.at[j]], buf)
                pltpu.sync_copy(buf, acc_sh.at[dst_v.at[j]], add=True)
                return 0

            lax.fori_loop(0, rpt, body, 0)

        @pl.when(c == 0)
        def _():
            do_edges(y0_hbm)

        @pl.when(c == 1)
        def _():
            do_edges(y1_hbm)

        plsc.subcore_barrier()

        def wout(a_hbm):
            pltpu.sync_copy(acc_sh.at[pl.ds(s * stripe, stripe)],
                            a_hbm.at[pl.ds(s * stripe, stripe)])

        @pl.when(c == 0)
        def _():
            wout(a0_hbm)

        @pl.when(c == 1)
        def _():
            wout(a1_hbm)

    return k(y0, y1, src2, dst2)


def _bn_stats_kernel(y0, y1, acc0, acc1, degp, bc2):
    n, half = y0.shape
    dh = 2 * half
    blk = 512
    nw = degp.shape[0]

    def body(y0_ref, y1_ref, a0_ref, a1_ref, degp_ref, bc_ref, h_ref, st_ref):
        i = pl.program_id(0)
        deg = jnp.sum(degp_ref[...], axis=0) + 1.0
        dinv = lax.rsqrt(deg)[:, None]
        h0 = jnp.maximum(dinv * (y0_ref[...] + a0_ref[...]) + bc_ref[0:1, :], 0.0)
        h1 = jnp.maximum(dinv * (y1_ref[...] + a1_ref[...]) + bc_ref[1:2, :], 0.0)
        h_ref[:, 0:half] = h0
        h_ref[:, half:dh] = h1

        @pl.when(i == 0)
        def _():
            st_ref[...] = jnp.zeros_like(st_ref)

        row_s = jnp.concatenate([jnp.sum(h0, axis=0), jnp.sum(h1, axis=0)])
        row_q = jnp.concatenate(
            [jnp.sum(h0 * h0, axis=0), jnp.sum(h1 * h1, axis=0)])
        st_ref[0:1, :] += row_s[None, :]
        st_ref[1:2, :] += row_q[None, :]

    return pl.pallas_call(
        body,
        grid=(n // blk,),
        in_specs=[
            pl.BlockSpec((blk, half), lambda i: (i, 0)),
            pl.BlockSpec((blk, half), lambda i: (i, 0)),
            pl.BlockSpec((blk, half), lambda i: (i, 0)),
            pl.BlockSpec((blk, half), lambda i: (i, 0)),
            pl.BlockSpec((nw, blk), lambda i: (0, i)),
            pl.BlockSpec((2, half), lambda i: (0, 0)),
        ],
        out_specs=[
            pl.BlockSpec((blk, dh), lambda i: (i, 0)),
            pl.BlockSpec((8, dh), lambda i: (0, 0)),
        ],
        out_shape=[
            jax.ShapeDtypeStruct((n, dh), jnp.float32),
            jax.ShapeDtypeStruct((8, dh), jnp.float32),
        ],
    )(y0, y1, acc0, acc1, degp, bc2)


def _fc_kernel(hg, st, gamma1, beta1, fc_w, fcb1, n_total):
    g, d4 = hg.shape
    dh = d4 // 4
    ncls = fc_w.shape[0]
    blk = 256

    def body(hg_ref, st_ref, ga_ref, be_ref, fw_ref, fb_ref, out_ref):
        inv_n = 1.0 / float(n_total)
        mean = st_ref[0:1, :] * inv_n
        ex2 = st_ref[1:2, :] * inv_n
        var = ex2 - mean * mean
        rstd = lax.rsqrt(var + 1e-5)
        scale = ga_ref[0:1, :] * rstd
        off = be_ref[0:1, :] - mean * scale
        scale4 = jnp.concatenate([scale] * 4, axis=1)
        off4 = jnp.concatenate([off] * 4, axis=1)
        hgn = hg_ref[...] * scale4 + off4
        logits = lax.dot_general(
            hgn, fw_ref[...], (((1,), (1,)), ((), ())),
            preferred_element_type=jnp.float32) + fb_ref[0:1, :]
        m = jnp.max(logits, axis=1, keepdims=True)
        lse = m + jnp.log(jnp.sum(jnp.exp(logits - m), axis=1, keepdims=True))
        ls = logits - lse
        out_ref[...] = jnp.broadcast_to(
            ls[:, None, :], (blk, 4, ncls)).reshape(4 * blk, ncls)

    return pl.pallas_call(
        body,
        grid=(g // blk,),
        in_specs=[
            pl.BlockSpec((blk, d4), lambda i: (i, 0)),
            pl.BlockSpec((8, dh), lambda i: (0, 0)),
            pl.BlockSpec((1, dh), lambda i: (0, 0)),
            pl.BlockSpec((1, dh), lambda i: (0, 0)),
            pl.BlockSpec((ncls, d4), lambda i: (0, 0)),
            pl.BlockSpec((1, ncls), lambda i: (0, 0)),
        ],
        out_specs=pl.BlockSpec((4 * blk, ncls), lambda i: (i, 0)),
        out_shape=jax.ShapeDtypeStruct((4 * g, ncls), jnp.float32),
    )(hg, st, gamma1, beta1, fc_w, fcb1)


def kernel(x, edge_index, num_graphs, W, b_conv, gamma, beta, fc_W, fc_b):
    del num_graphs
    n, _ = x.shape
    dh = W.shape[1]
    src = edge_index[0]
    dst = edge_index[1]
    e = src.shape[0]

    degp = _deg_kernel(dst, n)
    y0, y1 = _matmul_kernel(x, W, degp)
    src2 = src.reshape(e // 128, 128)
    dst2 = dst.reshape(e // 128, 128)
    acc0, acc1 = _scatter_kernel(y0, y1, src2, dst2, n)
    bc2 = b_conv.reshape(2, dh // 2)
    h, st = _bn_stats_kernel(y0, y1, acc0, acc1, degp, bc2)
    hg = h.reshape(n // 4, 4 * dh)
    return _fc_kernel(hg, st, gamma.reshape(1, dh), beta.reshape(1, dh),
                      fc_W, fc_b.reshape(1, -1), n)


# speedup vs baseline: 8.9946x; 1.1620x over previous
import functools

import jax
import jax.numpy as jnp
from jax import lax
from jax.experimental import pallas as pl
from jax.experimental.pallas import tpu as pltpu
from jax.experimental.pallas import tpu_sc as plsc

NC, NS, LANES = 2, 16, 16


def _deg_kernel(dst, n_nodes):
    e = dst.shape[0]
    nw = NC * NS
    ept = e // nw
    mesh = plsc.VectorSubcoreMesh(core_axis_name="c", subcore_axis_name="s")

    @functools.partial(
        pl.kernel,
        out_type=jax.ShapeDtypeStruct((nw, n_nodes), jnp.float32),
        mesh=mesh,
        scratch_types=[
            pltpu.VMEM((n_nodes,), jnp.float32),
            pltpu.VMEM((ept,), jnp.int32),
        ],
        compiler_params=pltpu.CompilerParams(needs_layout_passes=False),
    )
    def k(dst_hbm, out_hbm, deg_l, dst_v):
        c = lax.axis_index("c")
        s = lax.axis_index("s")
        wid = c * NS + s

        def zero(i, _):
            deg_l[pl.ds(i * LANES, LANES)] = jnp.zeros((LANES,), jnp.float32)
            return 0

        lax.fori_loop(0, n_nodes // LANES, zero, 0)

        pltpu.sync_copy(dst_hbm.at[pl.ds(wid * ept, ept)], dst_v)
        ones = jnp.ones((LANES,), jnp.float32)

        def acc(g, _):
            idx = dst_v[pl.ds(g * LANES, LANES)]
            plsc.addupdate_scatter(deg_l, [idx], ones)
            return 0

        lax.fori_loop(0, ept // LANES, acc, 0)
        pltpu.sync_copy(deg_l, out_hbm.at[wid])

    return k(dst)


def _matmul_kernel(x, w, degp):
    n, d_in = x.shape
    dh = w.shape[1]
    half = dh // 2
    blk = 512
    nw = degp.shape[0]

    def body(x_ref, w_ref, degp_ref, y_ref):
        deg = jnp.sum(degp_ref[...], axis=0) + 1.0
        dinv = lax.rsqrt(deg)
        y = jnp.dot(x_ref[...], w_ref[...], preferred_element_type=jnp.float32)
        y = y * dinv[:, None]
        y_ref[0] = y[:, :half]
        y_ref[1] = y[:, half:]

    return pl.pallas_call(
        body,
        grid=(n // blk,),
        in_specs=[
            pl.BlockSpec((blk, d_in), lambda i: (i, 0)),
            pl.BlockSpec((d_in, dh), lambda i: (0, 0)),
            pl.BlockSpec((nw, blk), lambda i: (0, i)),
        ],
        out_specs=pl.BlockSpec((2, blk, half), lambda i: (0, i, 0)),
        out_shape=jax.ShapeDtypeStruct((2, n, half), jnp.float32),
    )(x, w, degp)


def _scatter_kernel(ycat, src2, dst2, n_nodes):
    nrows = src2.shape[0]
    rpt = nrows // NS
    stripe = n_nodes // NS
    mesh = plsc.VectorSubcoreMesh(core_axis_name="c", subcore_axis_name="s")

    @functools.partial(
        pl.kernel,
        out_type=jax.ShapeDtypeStruct((2 * n_nodes, 128), jnp.float32),
        mesh=mesh,
        scratch_types=[
            pltpu.VMEM((rpt, 128), jnp.int32),
            pltpu.VMEM((rpt, 128), jnp.int32),
            pltpu.VMEM((128, 128), jnp.float32),
            pltpu.VMEM((128, 128), jnp.float32),
            pltpu.SemaphoreType.DMA,
            pltpu.SemaphoreType.DMA,
            pltpu.VMEM_SHARED((n_nodes, 128), jnp.float32),
        ],
    )
    def k(y_hbm, src_hbm, dst_hbm, a_hbm,
          src_v, dst_v, buf0, buf1, sem0, sem1, acc_sh):
        c = lax.axis_index("c")
        s = lax.axis_index("s")
        bufs = [buf0, buf1]
        sems = [sem0, sem1]
        cbase = c * n_nodes

        pltpu.sync_copy(src_hbm.at[pl.ds(s * rpt, rpt)], src_v)
        pltpu.sync_copy(dst_hbm.at[pl.ds(s * rpt, rpt)], dst_v)

        def off(t, _):
            sl = (t // 8, pl.ds((t % 8) * LANES, LANES))
            src_v[sl] = src_v[sl] + cbase
            return 0

        lax.fori_loop(0, rpt * 8, off, 0)

        pltpu.sync_copy(y_hbm.at[pl.ds(cbase + s * stripe, stripe)],
                        acc_sh.at[pl.ds(s * stripe, stripe)])
        plsc.subcore_barrier()
        d = pltpu.async_copy(y_hbm.at[src_v.at[0]], bufs[0], sems[0])
        for j in range(rpt):
            if j + 1 < rpt:
                d_next = pltpu.async_copy(
                    y_hbm.at[src_v.at[j + 1]], bufs[(j + 1) % 2],
                    sems[(j + 1) % 2])
            d.wait()
            pltpu.sync_copy(bufs[j % 2], acc_sh.at[dst_v.at[j]], add=True)
            if j + 1 < rpt:
                d = d_next
        plsc.subcore_barrier()
        pltpu.sync_copy(acc_sh.at[pl.ds(s * stripe, stripe)],
                        a_hbm.at[pl.ds(cbase + s * stripe, stripe)])

    return k(ycat, src2, dst2)


def _bn_stats_kernel(acat, degp, bc2):
    n2, half = acat.shape
    n = n2 // 2
    dh = 2 * half
    blk = 512
    nw = degp.shape[0]

    def body(a0_ref, a1_ref, degp_ref, bc_ref, h_ref, st_ref):
        i = pl.program_id(0)
        deg = jnp.sum(degp_ref[...], axis=0) + 1.0
        dinv = lax.rsqrt(deg)[:, None]
        h0 = jnp.maximum(dinv * a0_ref[...] + bc_ref[0:1, :], 0.0)
        h1 = jnp.maximum(dinv * a1_ref[...] + bc_ref[1:2, :], 0.0)
        h_ref[:, 0:half] = h0
        h_ref[:, half:dh] = h1

        @pl.when(i == 0)
        def _():
            st_ref[...] = jnp.zeros_like(st_ref)

        row_s = jnp.concatenate([jnp.sum(h0, axis=0), jnp.sum(h1, axis=0)])
        row_q = jnp.concatenate(
            [jnp.sum(h0 * h0, axis=0), jnp.sum(h1 * h1, axis=0)])
        st_ref[0:1, :] += row_s[None, :]
        st_ref[1:2, :] += row_q[None, :]

    return pl.pallas_call(
        body,
        grid=(n // blk,),
        in_specs=[
            pl.BlockSpec((blk, half), lambda i: (i, 0)),
            pl.BlockSpec((blk, half), lambda i: (n // blk + i, 0)),
            pl.BlockSpec((nw, blk), lambda i: (0, i)),
            pl.BlockSpec((2, half), lambda i: (0, 0)),
        ],
        out_specs=[
            pl.BlockSpec((blk, dh), lambda i: (i, 0)),
            pl.BlockSpec((8, dh), lambda i: (0, 0)),
        ],
        out_shape=[
            jax.ShapeDtypeStruct((n, dh), jnp.float32),
            jax.ShapeDtypeStruct((8, dh), jnp.float32),
        ],
    )(acat, acat, degp, bc2)


def _fc_kernel(hg, st, gamma1, beta1, fc_w, fcb1, n_total):
    g, d4 = hg.shape
    dh = d4 // 4
    ncls = fc_w.shape[0]
    blk = 256

    def body(hg_ref, st_ref, ga_ref, be_ref, fw_ref, fb_ref, out_ref):
        inv_n = 1.0 / float(n_total)
        mean = st_ref[0:1, :] * inv_n
        ex2 = st_ref[1:2, :] * inv_n
        var = ex2 - mean * mean
        rstd = lax.rsqrt(var + 1e-5)
        scale = ga_ref[0:1, :] * rstd
        off = be_ref[0:1, :] - mean * scale
        scale4 = jnp.concatenate([scale] * 4, axis=1)
        off4 = jnp.concatenate([off] * 4, axis=1)
        hgn = hg_ref[...] * scale4 + off4
        logits = lax.dot_general(
            hgn, fw_ref[...], (((1,), (1,)), ((), ())),
            preferred_element_type=jnp.float32) + fb_ref[0:1, :]
        m = jnp.max(logits, axis=1, keepdims=True)
        lse = m + jnp.log(jnp.sum(jnp.exp(logits - m), axis=1, keepdims=True))
        ls = logits - lse
        out_ref[...] = jnp.broadcast_to(
            ls[:, None, :], (blk, 4, ncls)).reshape(4 * blk, ncls)

    return pl.pallas_call(
        body,
        grid=(g // blk,),
        in_specs=[
            pl.BlockSpec((blk, d4), lambda i: (i, 0)),
            pl.BlockSpec((8, dh), lambda i: (0, 0)),
            pl.BlockSpec((1, dh), lambda i: (0, 0)),
            pl.BlockSpec((1, dh), lambda i: (0, 0)),
            pl.BlockSpec((ncls, d4), lambda i: (0, 0)),
            pl.BlockSpec((1, ncls), lambda i: (0, 0)),
        ],
        out_specs=pl.BlockSpec((4 * blk, ncls), lambda i: (i, 0)),
        out_shape=jax.ShapeDtypeStruct((4 * g, ncls), jnp.float32),
    )(hg, st, gamma1, beta1, fc_w, fcb1)


def kernel(x, edge_index, num_graphs, W, b_conv, gamma, beta, fc_W, fc_b):
    del num_graphs
    n, _ = x.shape
    dh = W.shape[1]
    src = edge_index[0]
    dst = edge_index[1]
    e = src.shape[0]

    degp = _deg_kernel(dst, n)
    ycat = _matmul_kernel(x, W, degp).reshape(2 * n, dh // 2)
    src2 = src.reshape(e // 128, 128)
    dst2 = dst.reshape(e // 128, 128)
    acat = _scatter_kernel(ycat, src2, dst2, n)
    bc2 = b_conv.reshape(2, dh // 2)
    h, st = _bn_stats_kernel(acat, degp, bc2)
    hg = h.reshape(n // 4, 4 * dh)
    return _fc_kernel(hg, st, gamma.reshape(1, dh), beta.reshape(1, dh),
                      fc_W, fc_b.reshape(1, -1), n)


# speedup vs baseline: 10.8906x; 1.2108x over previous
import functools

import jax
import jax.numpy as jnp
from jax import lax
from jax.experimental import pallas as pl
from jax.experimental.pallas import tpu as pltpu
from jax.experimental.pallas import tpu_sc as plsc

NC, NS, LANES = 2, 16, 16


def _deg_kernel(dst, n_nodes):
    e = dst.shape[0]
    nw = NC * NS
    ept = e // nw
    mesh = plsc.VectorSubcoreMesh(core_axis_name="c", subcore_axis_name="s")

    @functools.partial(
        pl.kernel,
        out_type=jax.ShapeDtypeStruct((nw, n_nodes), jnp.float32),
        mesh=mesh,
        scratch_types=[
            pltpu.VMEM((n_nodes,), jnp.float32),
            pltpu.VMEM((ept,), jnp.int32),
        ],
        compiler_params=pltpu.CompilerParams(needs_layout_passes=False),
    )
    def k(dst_hbm, out_hbm, deg_l, dst_v):
        c = lax.axis_index("c")
        s = lax.axis_index("s")
        wid = c * NS + s

        def zero(i, _):
            deg_l[pl.ds(i * LANES, LANES)] = jnp.zeros((LANES,), jnp.float32)
            return 0

        lax.fori_loop(0, n_nodes // LANES, zero, 0)

        pltpu.sync_copy(dst_hbm.at[pl.ds(wid * ept, ept)], dst_v)
        ones = jnp.ones((LANES,), jnp.float32)

        def acc(g, _):
            idx = dst_v[pl.ds(g * LANES, LANES)]
            plsc.addupdate_scatter(deg_l, [idx], ones)
            return 0

        lax.fori_loop(0, ept // LANES, acc, 0)
        pltpu.sync_copy(deg_l, out_hbm.at[wid])

    return k(dst)


def _matmul_kernel(x, w, degp):
    n, d_in = x.shape
    dh = w.shape[1]
    half = dh // 2
    blk = 512
    nw = degp.shape[0]

    def body(x_ref, w_ref, degp_ref, y_ref):
        deg = jnp.sum(degp_ref[...], axis=0) + 1.0
        dinv = lax.rsqrt(deg)
        y = jnp.dot(x_ref[...], w_ref[...], preferred_element_type=jnp.float32)
        y = y * dinv[:, None]
        y_ref[0] = y[:, :half]
        y_ref[1] = y[:, half:]

    return pl.pallas_call(
        body,
        grid=(n // blk,),
        in_specs=[
            pl.BlockSpec((blk, d_in), lambda i: (i, 0)),
            pl.BlockSpec((d_in, dh), lambda i: (0, 0)),
            pl.BlockSpec((nw, blk), lambda i: (0, i)),
        ],
        out_specs=pl.BlockSpec((2, blk, half), lambda i: (0, i, 0)),
        out_shape=jax.ShapeDtypeStruct((2, n, half), jnp.float32),
    )(x, w, degp)


def _scatter_kernel(ycat, src2, dst2, n_nodes):
    nrows = src2.shape[0]
    rpt = nrows // NS
    stripe = n_nodes // NS
    mesh = plsc.VectorSubcoreMesh(core_axis_name="c", subcore_axis_name="s")

    @functools.partial(
        pl.kernel,
        out_type=jax.ShapeDtypeStruct((2 * n_nodes, 128), jnp.float32),
        mesh=mesh,
        scratch_types=[
            pltpu.VMEM((rpt, 128), jnp.int32),
            pltpu.VMEM((rpt, 128), jnp.int32),
            pltpu.VMEM((128, 128), jnp.float32),
            pltpu.VMEM((128, 128), jnp.float32),
            pltpu.SemaphoreType.DMA,
            pltpu.SemaphoreType.DMA,
            pltpu.VMEM_SHARED((n_nodes, 128), jnp.float32),
        ],
    )
    def k(y_hbm, src_hbm, dst_hbm, a_hbm,
          src_v, dst_v, buf0, buf1, sem0, sem1, acc_sh):
        c = lax.axis_index("c")
        s = lax.axis_index("s")
        bufs = [buf0, buf1]
        sems = [sem0, sem1]
        cbase = c * n_nodes

        pltpu.sync_copy(src_hbm.at[pl.ds(s * rpt, rpt)], src_v)
        pltpu.sync_copy(dst_hbm.at[pl.ds(s * rpt, rpt)], dst_v)

        def off(t, _):
            sl = (t // 8, pl.ds((t % 8) * LANES, LANES))
            src_v[sl] = src_v[sl] + cbase
            return 0

        lax.fori_loop(0, rpt * 8, off, 0)

        pltpu.sync_copy(y_hbm.at[pl.ds(cbase + s * stripe, stripe)],
                        acc_sh.at[pl.ds(s * stripe, stripe)])
        plsc.subcore_barrier()
        d = pltpu.async_copy(y_hbm.at[src_v.at[0]], bufs[0], sems[0])
        for j in range(rpt):
            if j + 1 < rpt:
                d_next = pltpu.async_copy(
                    y_hbm.at[src_v.at[j + 1]], bufs[(j + 1) % 2],
                    sems[(j + 1) % 2])
            d.wait()
            pltpu.sync_copy(bufs[j % 2], acc_sh.at[dst_v.at[j]], add=True)
            if j + 1 < rpt:
                d = d_next
        plsc.subcore_barrier()
        pltpu.sync_copy(acc_sh.at[pl.ds(s * stripe, stripe)],
                        a_hbm.at[pl.ds(cbase + s * stripe, stripe)])

    return k(ycat, src2, dst2)


def _bn_stats_kernel(acat, degp, bc2):
    n2, half = acat.shape
    n = n2 // 2
    dh = 2 * half
    blk = 512
    nw = degp.shape[0]

    def body(a0_ref, a1_ref, degp_ref, bc_ref, h_ref, st_ref):
        i = pl.program_id(0)
        deg = jnp.sum(degp_ref[...], axis=0) + 1.0
        dinv = lax.rsqrt(deg)[:, None]
        h0 = jnp.maximum(dinv * a0_ref[...] + bc_ref[0:1, :], 0.0)
        h1 = jnp.maximum(dinv * a1_ref[...] + bc_ref[1:2, :], 0.0)
        hcat = jnp.concatenate([h0, h1], axis=1)
        h_ref[...] = hcat.reshape(blk // 4, 4 * dh)

        @pl.when(i == 0)
        def _():
            st_ref[...] = jnp.zeros_like(st_ref)

        row_s = jnp.concatenate([jnp.sum(h0, axis=0), jnp.sum(h1, axis=0)])
        row_q = jnp.concatenate(
            [jnp.sum(h0 * h0, axis=0), jnp.sum(h1 * h1, axis=0)])
        st_ref[0:1, :] += row_s[None, :]
        st_ref[1:2, :] += row_q[None, :]

    return pl.pallas_call(
        body,
        grid=(n // blk,),
        in_specs=[
            pl.BlockSpec((blk, half), lambda i: (i, 0)),
            pl.BlockSpec((blk, half), lambda i: (n // blk + i, 0)),
            pl.BlockSpec((nw, blk), lambda i: (0, i)),
            pl.BlockSpec((2, half), lambda i: (0, 0)),
        ],
        out_specs=[
            pl.BlockSpec((blk // 4, 4 * dh), lambda i: (i, 0)),
            pl.BlockSpec((8, dh), lambda i: (0, 0)),
        ],
        out_shape=[
            jax.ShapeDtypeStruct((n // 4, 4 * dh), jnp.float32),
            jax.ShapeDtypeStruct((8, dh), jnp.float32),
        ],
    )(acat, acat, degp, bc2)


def _fc_kernel(hg, st, gamma1, beta1, fc_w, fcb1, n_total):
    g, d4 = hg.shape
    dh = d4 // 4
    ncls = fc_w.shape[0]
    blk = 256

    def body(hg_ref, st_ref, ga_ref, be_ref, fw_ref, fb_ref, out_ref):
        inv_n = 1.0 / float(n_total)
        mean = st_ref[0:1, :] * inv_n
        ex2 = st_ref[1:2, :] * inv_n
        var = ex2 - mean * mean
        rstd = lax.rsqrt(var + 1e-5)
        scale = ga_ref[0:1, :] * rstd
        off = be_ref[0:1, :] - mean * scale
        scale4 = jnp.concatenate([scale] * 4, axis=1)
        off4 = jnp.concatenate([off] * 4, axis=1)
        hgn = hg_ref[...] * scale4 + off4
        ltT = lax.dot_general(
            fw_ref[...], hgn, (((1,), (1,)), ((), ())),
            preferred_element_type=jnp.float32) + fb_ref[...]
        m = jnp.max(ltT, axis=0, keepdims=True)
        lse = m + jnp.log(jnp.sum(jnp.exp(ltT - m), axis=0, keepdims=True))
        lsT = ltT - lse
        gsrc = lax.broadcasted_iota(jnp.int32, (blk, 4 * blk), 0)
        gdst = lax.broadcasted_iota(jnp.int32, (blk, 4 * blk), 1) // 4
        rep = (gsrc == gdst).astype(jnp.float32)
        out_ref[...] = lax.dot_general(
            lsT, rep, (((1,), (0,)), ((), ())),
            preferred_element_type=jnp.float32)

    return pl.pallas_call(
        body,
        grid=(g // blk,),
        in_specs=[
            pl.BlockSpec((blk, d4), lambda i: (i, 0)),
            pl.BlockSpec((8, dh), lambda i: (0, 0)),
            pl.BlockSpec((1, dh), lambda i: (0, 0)),
            pl.BlockSpec((1, dh), lambda i: (0, 0)),
            pl.BlockSpec((ncls, d4), lambda i: (0, 0)),
            pl.BlockSpec((ncls, 1), lambda i: (0, 0)),
        ],
        out_specs=pl.BlockSpec((ncls, 4 * blk), lambda i: (0, i)),
        out_shape=jax.ShapeDtypeStruct((ncls, 4 * g), jnp.float32),
    )(hg, st, gamma1, beta1, fc_w, fcb1)


def kernel(x, edge_index, num_graphs, W, b_conv, gamma, beta, fc_W, fc_b):
    del num_graphs
    n, _ = x.shape
    dh = W.shape[1]
    src = edge_index[0]
    dst = edge_index[1]
    e = src.shape[0]

    degp = _deg_kernel(dst, n)
    ycat = _matmul_kernel(x, W, degp).reshape(2 * n, dh // 2)
    src2 = src.reshape(e // 128, 128)
    dst2 = dst.reshape(e // 128, 128)
    acat = _scatter_kernel(ycat, src2, dst2, n)
    bc2 = b_conv.reshape(2, dh // 2)
    hg, st = _bn_stats_kernel(acat, degp, bc2)
    outT = _fc_kernel(hg, st, gamma.reshape(1, dh), beta.reshape(1, dh),
                      fc_W, fc_b.reshape(-1, 1), n)
    return outT.T


# speedup vs baseline: 11.3309x; 1.0404x over previous
import functools

import jax
import jax.numpy as jnp
from jax import lax
from jax.experimental import pallas as pl
from jax.experimental.pallas import tpu as pltpu
from jax.experimental.pallas import tpu_sc as plsc

NC, NS, LANES = 2, 16, 16


def _deg_kernel(dst, n_nodes):
    e = dst.shape[0]
    nw = NC * NS
    ept = e // nw
    mesh = plsc.VectorSubcoreMesh(core_axis_name="c", subcore_axis_name="s")

    @functools.partial(
        pl.kernel,
        out_type=jax.ShapeDtypeStruct((nw, n_nodes), jnp.float32),
        mesh=mesh,
        scratch_types=[
            pltpu.VMEM((n_nodes,), jnp.float32),
            pltpu.VMEM((ept,), jnp.int32),
        ],
        compiler_params=pltpu.CompilerParams(needs_layout_passes=False),
    )
    def k(dst_hbm, out_hbm, deg_l, dst_v):
        c = lax.axis_index("c")
        s = lax.axis_index("s")
        wid = c * NS + s

        def zero(i, _):
            deg_l[pl.ds(i * LANES, LANES)] = jnp.zeros((LANES,), jnp.float32)
            return 0

        lax.fori_loop(0, n_nodes // LANES, zero, 0)

        pltpu.sync_copy(dst_hbm.at[pl.ds(wid * ept, ept)], dst_v)
        ones = jnp.ones((LANES,), jnp.float32)

        def acc(g, _):
            idx = dst_v[pl.ds(g * LANES, LANES)]
            plsc.addupdate_scatter(deg_l, [idx], ones)
            return 0

        lax.fori_loop(0, ept // LANES, acc, 0)
        pltpu.sync_copy(deg_l, out_hbm.at[wid])

    return k(dst)


def _matmul_kernel(x, w, degp):
    n, d_in = x.shape
    dh = w.shape[1]
    half = dh // 2
    blk = 512
    nw = degp.shape[0]

    def body(x_ref, w_ref, degp_ref, y_ref):
        deg = jnp.sum(degp_ref[...], axis=0) + 1.0
        dinv = lax.rsqrt(deg)
        y = jnp.dot(x_ref[...], w_ref[...], preferred_element_type=jnp.float32)
        y = y * dinv[:, None]
        y_ref[0] = y[:, :half]
        y_ref[1] = y[:, half:]

    return pl.pallas_call(
        body,
        grid=(n // blk,),
        in_specs=[
            pl.BlockSpec((blk, d_in), lambda i: (i, 0)),
            pl.BlockSpec((d_in, dh), lambda i: (0, 0)),
            pl.BlockSpec((nw, blk), lambda i: (0, i)),
        ],
        out_specs=pl.BlockSpec((2, blk, half), lambda i: (0, i, 0)),
        out_shape=jax.ShapeDtypeStruct((2, n, half), jnp.float32),
    )(x, w, degp)


def _scatter_kernel(ycat, src2, dst2, n_nodes):
    nrows = src2.shape[0]
    rpt = nrows // NS
    stripe = n_nodes // NS
    mesh = plsc.VectorSubcoreMesh(core_axis_name="c", subcore_axis_name="s")

    @functools.partial(
        pl.kernel,
        out_type=jax.ShapeDtypeStruct((2 * n_nodes, 128), jnp.float32),
        mesh=mesh,
        scratch_types=[
            pltpu.VMEM((rpt, 128), jnp.int32),
            pltpu.VMEM((rpt, 128), jnp.int32),
            pltpu.VMEM((128, 128), jnp.float32),
            pltpu.VMEM((128, 128), jnp.float32),
            pltpu.SemaphoreType.DMA,
            pltpu.SemaphoreType.DMA,
            pltpu.VMEM_SHARED((n_nodes, 128), jnp.float32),
        ],
    )
    def k(y_hbm, src_hbm, dst_hbm, a_hbm,
          src_v, dst_v, buf0, buf1, sem0, sem1, acc_sh):
        c = lax.axis_index("c")
        s = lax.axis_index("s")
        bufs = [buf0, buf1]
        sems = [sem0, sem1]
        cbase = c * n_nodes

        pltpu.sync_copy(src_hbm.at[pl.ds(s * rpt, rpt)], src_v)
        pltpu.sync_copy(dst_hbm.at[pl.ds(s * rpt, rpt)], dst_v)

        def off(t, _):
            sl = (t // 8, pl.ds((t % 8) * LANES, LANES))
            src_v[sl] = src_v[sl] + cbase
            return 0

        lax.fori_loop(0, rpt * 8, off, 0)

        pltpu.sync_copy(y_hbm.at[pl.ds(cbase + s * stripe, stripe)],
                        acc_sh.at[pl.ds(s * stripe, stripe)])
        plsc.subcore_barrier()
        d = pltpu.async_copy(y_hbm.at[src_v.at[0]], bufs[0], sems[0])
        for j in range(rpt):
            if j + 1 < rpt:
                d_next = pltpu.async_copy(
                    y_hbm.at[src_v.at[j + 1]], bufs[(j + 1) % 2],
                    sems[(j + 1) % 2])
            d.wait()
            pltpu.sync_copy(bufs[j % 2], acc_sh.at[dst_v.at[j]], add=True)
            if j + 1 < rpt:
                d = d_next
        plsc.subcore_barrier()
        pltpu.sync_copy(acc_sh.at[pl.ds(s * stripe, stripe)],
                        a_hbm.at[pl.ds(cbase + s * stripe, stripe)])

    return k(ycat, src2, dst2)


def _bn_fc_kernel(acat, degp, bc2, gamma1, beta1, fc_w, fcb1):
    n2, half = acat.shape
    n = n2 // 2
    dh = 2 * half
    d4 = 4 * dh
    blk = 512
    gblk = 256
    g_all = n // 4
    p1 = n // blk
    p2 = g_all // gblk
    nw = degp.shape[0]
    ncls = fc_w.shape[0]
    inv_n = 1.0 / float(n)

    def body(a0_ref, a1_ref, degp_ref, bc_ref, ga_ref, be_ref, fw_ref,
             fb_ref, out_ref, hg_s, st_s):
        i = pl.program_id(0)

        @pl.when(i == 0)
        def _():
            st_s[...] = jnp.zeros_like(st_s)

        @pl.when(i < p1)
        def _():
            deg = jnp.sum(degp_ref[...], axis=0) + 1.0
            dinv = lax.rsqrt(deg)[:, None]
            h0 = jnp.maximum(dinv * a0_ref[...] + bc_ref[0:1, :], 0.0)
            h1 = jnp.maximum(dinv * a1_ref[...] + bc_ref[1:2, :], 0.0)
            hcat = jnp.concatenate([h0, h1], axis=1)
            hg_s[pl.ds(i * (blk // 4), blk // 4), :] = hcat.reshape(
                blk // 4, d4)
            row_s = jnp.concatenate(
                [jnp.sum(h0, axis=0), jnp.sum(h1, axis=0)])
            row_q = jnp.concatenate(
                [jnp.sum(h0 * h0, axis=0), jnp.sum(h1 * h1, axis=0)])
            st_s[0:1, :] += row_s[None, :]
            st_s[1:2, :] += row_q[None, :]

        @pl.when(i >= p1)
        def _():
            j = i - p1
            mean = st_s[0:1, :] * inv_n
            ex2 = st_s[1:2, :] * inv_n
            var = ex2 - mean * mean
            rstd = lax.rsqrt(var + 1e-5)
            scale = ga_ref[0:1, :] * rstd
            off = be_ref[0:1, :] - mean * scale
            scale4 = jnp.concatenate([scale] * 4, axis=1)
            off4 = jnp.concatenate([off] * 4, axis=1)
            hgn = hg_s[pl.ds(j * gblk, gblk), :] * scale4 + off4
            ltT = lax.dot_general(
                fw_ref[...], hgn, (((1,), (1,)), ((), ())),
                preferred_element_type=jnp.float32) + fb_ref[...]
            m = jnp.max(ltT, axis=0, keepdims=True)
            lse = m + jnp.log(
                jnp.sum(jnp.exp(ltT - m), axis=0, keepdims=True))
            lsT = ltT - lse
            gsrc = lax.broadcasted_iota(jnp.int32, (gblk, 4 * gblk), 0)
            gdst = lax.broadcasted_iota(jnp.int32, (gblk, 4 * gblk), 1) // 4
            rep = (gsrc == gdst).astype(jnp.float32)
            out_ref[...] = lax.dot_general(
                lsT, rep, (((1,), (0,)), ((), ())),
                preferred_element_type=jnp.float32)

    return pl.pallas_call(
        body,
        grid=(p1 + p2,),
        in_specs=[
            pl.BlockSpec((blk, half), lambda i: (jnp.minimum(i, p1 - 1), 0)),
            pl.BlockSpec((blk, half),
                         lambda i: (p1 + jnp.minimum(i, p1 - 1), 0)),
            pl.BlockSpec((nw, blk), lambda i: (0, jnp.minimum(i, p1 - 1))),
            pl.BlockSpec((2, half), lambda i: (0, 0)),
            pl.BlockSpec((1, dh), lambda i: (0, 0)),
            pl.BlockSpec((1, dh), lambda i: (0, 0)),
            pl.BlockSpec((ncls, d4), lambda i: (0, 0)),
            pl.BlockSpec((ncls, 1), lambda i: (0, 0)),
        ],
        out_specs=pl.BlockSpec(
            (ncls, 4 * gblk), lambda i: (0, jnp.maximum(i - p1, 0))),
        out_shape=jax.ShapeDtypeStruct((ncls, n), jnp.float32),
        scratch_shapes=[
            pltpu.VMEM((g_all, d4), jnp.float32),
            pltpu.VMEM((8, dh), jnp.float32),
        ],
    )(acat, acat, degp, bc2, gamma1, beta1, fc_w, fcb1)


def kernel(x, edge_index, num_graphs, W, b_conv, gamma, beta, fc_W, fc_b):
    del num_graphs
    n, _ = x.shape
    dh = W.shape[1]
    src = edge_index[0]
    dst = edge_index[1]
    e = src.shape[0]

    degp = _deg_kernel(dst, n)
    ycat = _matmul_kernel(x, W, degp).reshape(2 * n, dh // 2)
    src2 = src.reshape(e // 128, 128)
    dst2 = dst.reshape(e // 128, 128)
    acat = _scatter_kernel(ycat, src2, dst2, n)
    bc2 = b_conv.reshape(2, dh // 2)
    outT = _bn_fc_kernel(acat, degp, bc2, gamma.reshape(1, dh),
                         beta.reshape(1, dh), fc_W, fc_b.reshape(-1, 1))
    return outT.T


# speedup vs baseline: 12.0152x; 1.0604x over previous
import functools

import jax
import jax.numpy as jnp
from jax import lax
from jax.experimental import pallas as pl
from jax.experimental.pallas import tpu as pltpu
from jax.experimental.pallas import tpu_sc as plsc

NC, NS, LANES = 2, 16, 16


def _deg_kernel(dst, n_nodes):
    e = dst.shape[0]
    nw = NC * NS
    ept = e // nw
    mesh = plsc.VectorSubcoreMesh(core_axis_name="c", subcore_axis_name="s")

    @functools.partial(
        pl.kernel,
        out_type=jax.ShapeDtypeStruct((nw, n_nodes), jnp.float32),
        mesh=mesh,
        scratch_types=[
            pltpu.VMEM((n_nodes,), jnp.float32),
            pltpu.VMEM((ept,), jnp.int32),
        ],
        compiler_params=pltpu.CompilerParams(needs_layout_passes=False),
    )
    def k(dst_hbm, out_hbm, deg_l, dst_v):
        c = lax.axis_index("c")
        s = lax.axis_index("s")
        wid = c * NS + s

        def zero(i, _):
            deg_l[pl.ds(i * LANES, LANES)] = jnp.zeros((LANES,), jnp.float32)
            return 0

        lax.fori_loop(0, n_nodes // LANES, zero, 0)

        pltpu.sync_copy(dst_hbm.at[pl.ds(wid * ept, ept)], dst_v)
        ones = jnp.ones((LANES,), jnp.float32)

        def acc(g, _):
            idx = dst_v[pl.ds(g * LANES, LANES)]
            plsc.addupdate_scatter(deg_l, [idx], ones)
            return 0

        lax.fori_loop(0, ept // LANES, acc, 0)
        pltpu.sync_copy(deg_l, out_hbm.at[wid])

    return k(dst)


def _matmul_kernel(x, w, degp):
    n, d_in = x.shape
    dh = w.shape[1]
    half = dh // 2
    blk = 512
    nw = degp.shape[0]

    def body(x_ref, w_ref, degp_ref, y_ref):
        deg = jnp.sum(degp_ref[...], axis=0) + 1.0
        dinv = lax.rsqrt(deg)
        y = jnp.dot(x_ref[...], w_ref[...], preferred_element_type=jnp.float32)
        y = y * dinv[:, None]
        y_ref[0] = y[:, :half]
        y_ref[1] = y[:, half:]

    return pl.pallas_call(
        body,
        grid=(n // blk,),
        in_specs=[
            pl.BlockSpec((blk, d_in), lambda i: (i, 0)),
            pl.BlockSpec((d_in, dh), lambda i: (0, 0)),
            pl.BlockSpec((nw, blk), lambda i: (0, i)),
        ],
        out_specs=pl.BlockSpec((2, blk, half), lambda i: (0, i, 0)),
        out_shape=jax.ShapeDtypeStruct((2, n, half), jnp.float32),
    )(x, w, degp)


def _scatter_kernel(ycat, src2, dst2, n_nodes):
    nrows = src2.shape[0]
    rpt = nrows // NS
    stripe = n_nodes // NS
    mesh = plsc.VectorSubcoreMesh(core_axis_name="c", subcore_axis_name="s")

    @functools.partial(
        pl.kernel,
        out_type=jax.ShapeDtypeStruct((2 * n_nodes, 128), jnp.float32),
        mesh=mesh,
        scratch_types=[
            pltpu.VMEM((rpt, 128), jnp.int32),
            pltpu.VMEM((rpt, 128), jnp.int32),
            pltpu.VMEM((128, 128), jnp.float32),
            pltpu.VMEM((128, 128), jnp.float32),
            pltpu.VMEM((128, 128), jnp.float32),
            pltpu.SemaphoreType.DMA,
            pltpu.SemaphoreType.DMA,
            pltpu.SemaphoreType.DMA,
            pltpu.SemaphoreType.DMA,
            pltpu.VMEM_SHARED((n_nodes, 128), jnp.float32),
        ],
    )
    def k(y_hbm, src_hbm, dst_hbm, a_hbm,
          src_v, dst_v, buf0, buf1, buf2,
          gsem0, gsem1, ssem, seedsem, acc_sh):
        c = lax.axis_index("c")
        s = lax.axis_index("s")
        bufs = [buf0, buf1, buf2]
        gsems = [gsem0, gsem1]
        cbase = c * n_nodes

        seed = pltpu.async_copy(y_hbm.at[pl.ds(cbase + s * stripe, stripe)],
                                acc_sh.at[pl.ds(s * stripe, stripe)], seedsem)

        pltpu.sync_copy(src_hbm.at[pl.ds(s * rpt, rpt)], src_v)
        pltpu.sync_copy(dst_hbm.at[pl.ds(s * rpt, rpt)], dst_v)

        def off(t, _):
            sl = (t // 8, pl.ds((t % 8) * LANES, LANES))
            src_v[sl] = src_v[sl] + cbase
            return 0

        lax.fori_loop(0, rpt * 8, off, 0)

        gd = [None] * rpt
        sd = [None] * rpt
        gd[0] = pltpu.async_copy(y_hbm.at[src_v.at[0]], bufs[0], gsems[0])
        if rpt > 1:
            gd[1] = pltpu.async_copy(y_hbm.at[src_v.at[1]], bufs[1], gsems[1])
        seed.wait()
        plsc.subcore_barrier()
        for j in range(rpt):
            gd[j].wait()
            if j >= 1:
                sd[j - 1].wait()
            sd[j] = pltpu.async_copy(bufs[j % 3], acc_sh.at[dst_v.at[j]],
                                     ssem, add=True)
            if j + 2 < rpt:
                gd[j + 2] = pltpu.async_copy(
                    y_hbm.at[src_v.at[j + 2]], bufs[(j + 2) % 3],
                    gsems[j % 2])
        sd[rpt - 1].wait()
        plsc.subcore_barrier()
        pltpu.sync_copy(acc_sh.at[pl.ds(s * stripe, stripe)],
                        a_hbm.at[pl.ds(cbase + s * stripe, stripe)])

    return k(ycat, src2, dst2)


def _bn_fc_kernel(acat, degp, bc2, gamma1, beta1, fc_w, fcb1):
    n2, half = acat.shape
    n = n2 // 2
    dh = 2 * half
    d4 = 4 * dh
    blk = 512
    gblk = 256
    g_all = n // 4
    p1 = n // blk
    p2 = g_all // gblk
    nw = degp.shape[0]
    ncls = fc_w.shape[0]
    inv_n = 1.0 / float(n)

    def body(a0_ref, a1_ref, degp_ref, bc_ref, ga_ref, be_ref, fw_ref,
             fb_ref, out_ref, hg_s, st_s):
        i = pl.program_id(0)

        @pl.when(i == 0)
        def _():
            st_s[...] = jnp.zeros_like(st_s)

        @pl.when(i < p1)
        def _():
            deg = jnp.sum(degp_ref[...], axis=0) + 1.0
            dinv = lax.rsqrt(deg)[:, None]
            h0 = jnp.maximum(dinv * a0_ref[...] + bc_ref[0:1, :], 0.0)
            h1 = jnp.maximum(dinv * a1_ref[...] + bc_ref[1:2, :], 0.0)
            hcat = jnp.concatenate([h0, h1], axis=1)
            hg_s[pl.ds(i * (blk // 4), blk // 4), :] = hcat.reshape(
                blk // 4, d4)
            row_s = jnp.concatenate(
                [jnp.sum(h0, axis=0), jnp.sum(h1, axis=0)])
            row_q = jnp.concatenate(
                [jnp.sum(h0 * h0, axis=0), jnp.sum(h1 * h1, axis=0)])
            st_s[0:1, :] += row_s[None, :]
            st_s[1:2, :] += row_q[None, :]

        @pl.when(i >= p1)
        def _():
            j = i - p1
            mean = st_s[0:1, :] * inv_n
            ex2 = st_s[1:2, :] * inv_n
            var = ex2 - mean * mean
            rstd = lax.rsqrt(var + 1e-5)
            scale = ga_ref[0:1, :] * rstd
            off = be_ref[0:1, :] - mean * scale
            scale4 = jnp.concatenate([scale] * 4, axis=1)
            off4 = jnp.concatenate([off] * 4, axis=1)
            hgn = hg_s[pl.ds(j * gblk, gblk), :] * scale4 + off4
            ltT = lax.dot_general(
                fw_ref[...], hgn, (((1,), (1,)), ((), ())),
                preferred_element_type=jnp.float32) + fb_ref[...]
            m = jnp.max(ltT, axis=0, keepdims=True)
            lse = m + jnp.log(
                jnp.sum(jnp.exp(ltT - m), axis=0, keepdims=True))
            lsT = ltT - lse
            gsrc = lax.broadcasted_iota(jnp.int32, (gblk, 4 * gblk), 0)
            gdst = lax.broadcasted_iota(jnp.int32, (gblk, 4 * gblk), 1) // 4
            rep = (gsrc == gdst).astype(jnp.float32)
            out_ref[...] = lax.dot_general(
                lsT, rep, (((1,), (0,)), ((), ())),
                preferred_element_type=jnp.float32)

    return pl.pallas_call(
        body,
        grid=(p1 + p2,),
        in_specs=[
            pl.BlockSpec((blk, half), lambda i: (jnp.minimum(i, p1 - 1), 0)),
            pl.BlockSpec((blk, half),
                         lambda i: (p1 + jnp.minimum(i, p1 - 1), 0)),
            pl.BlockSpec((nw, blk), lambda i: (0, jnp.minimum(i, p1 - 1))),
            pl.BlockSpec((2, half), lambda i: (0, 0)),
            pl.BlockSpec((1, dh), lambda i: (0, 0)),
            pl.BlockSpec((1, dh), lambda i: (0, 0)),
            pl.BlockSpec((ncls, d4), lambda i: (0, 0)),
            pl.BlockSpec((ncls, 1), lambda i: (0, 0)),
        ],
        out_specs=pl.BlockSpec(
            (ncls, 4 * gblk), lambda i: (0, jnp.maximum(i - p1, 0))),
        out_shape=jax.ShapeDtypeStruct((ncls, n), jnp.float32),
        scratch_shapes=[
            pltpu.VMEM((g_all, d4), jnp.float32),
            pltpu.VMEM((8, dh), jnp.float32),
        ],
    )(acat, acat, degp, bc2, gamma1, beta1, fc_w, fcb1)


def kernel(x, edge_index, num_graphs, W, b_conv, gamma, beta, fc_W, fc_b):
    del num_graphs
    n, _ = x.shape
    dh = W.shape[1]
    src = edge_index[0]
    dst = edge_index[1]
    e = src.shape[0]

    degp = _deg_kernel(dst, n)
    ycat = _matmul_kernel(x, W, degp).reshape(2 * n, dh // 2)
    src2 = src.reshape(e // 128, 128)
    dst2 = dst.reshape(e // 128, 128)
    acat = _scatter_kernel(ycat, src2, dst2, n)
    bc2 = b_conv.reshape(2, dh // 2)
    outT = _bn_fc_kernel(acat, degp, bc2, gamma.reshape(1, dh),
                         beta.reshape(1, dh), fc_W, fc_b.reshape(-1, 1))
    return outT.T


# speedup vs baseline: 12.2477x; 1.0194x over previous
import functools

import jax
import jax.numpy as jnp
from jax import lax
from jax.experimental import pallas as pl
from jax.experimental.pallas import tpu as pltpu
from jax.experimental.pallas import tpu_sc as plsc

NC, NS, LANES = 2, 16, 16


def _deg_kernel(dst, n_nodes):
    e = dst.shape[0]
    nw = NC * NS
    ept = e // nw
    mesh = plsc.VectorSubcoreMesh(core_axis_name="c", subcore_axis_name="s")

    @functools.partial(
        pl.kernel,
        out_type=jax.ShapeDtypeStruct((nw, n_nodes), jnp.float32),
        mesh=mesh,
        scratch_types=[
            pltpu.VMEM((n_nodes,), jnp.float32),
            pltpu.VMEM((ept,), jnp.int32),
        ],
        compiler_params=pltpu.CompilerParams(needs_layout_passes=False),
    )
    def k(dst_hbm, out_hbm, deg_l, dst_v):
        c = lax.axis_index("c")
        s = lax.axis_index("s")
        wid = c * NS + s

        def zero(i, _):
            deg_l[pl.ds(i * LANES, LANES)] = jnp.zeros((LANES,), jnp.float32)
            return 0

        lax.fori_loop(0, n_nodes // LANES, zero, 0)

        pltpu.sync_copy(dst_hbm.at[pl.ds(wid * ept, ept)], dst_v)
        ones = jnp.ones((LANES,), jnp.float32)

        def acc(g, _):
            idx = dst_v[pl.ds(g * LANES, LANES)]
            plsc.addupdate_scatter(deg_l, [idx], ones)
            return 0

        lax.fori_loop(0, ept // LANES, acc, 0)
        pltpu.sync_copy(deg_l, out_hbm.at[wid])

    return k(dst)


def _matmul_kernel(x, w, degp):
    n, d_in = x.shape
    dh = w.shape[1]
    half = dh // 2
    blk = 1024
    nw = degp.shape[0]

    def body(x_ref, w_ref, degp_ref, y_ref):
        deg = jnp.sum(degp_ref[...], axis=0) + 1.0
        dinv = lax.rsqrt(deg)
        y = jnp.dot(x_ref[...], w_ref[...], preferred_element_type=jnp.float32)
        y = y * dinv[:, None]
        y_ref[0] = y[:, :half]
        y_ref[1] = y[:, half:]

    return pl.pallas_call(
        body,
        grid=(n // blk,),
        in_specs=[
            pl.BlockSpec((blk, d_in), lambda i: (i, 0)),
            pl.BlockSpec((d_in, dh), lambda i: (0, 0)),
            pl.BlockSpec((nw, blk), lambda i: (0, i)),
        ],
        out_specs=pl.BlockSpec((2, blk, half), lambda i: (0, i, 0)),
        out_shape=jax.ShapeDtypeStruct((2, n, half), jnp.float32),
    )(x, w, degp)


def _scatter_kernel(ycat, src2, dst2, n_nodes):
    nrows = src2.shape[0]
    rpt = nrows // NS
    stripe = n_nodes // NS
    mesh = plsc.VectorSubcoreMesh(core_axis_name="c", subcore_axis_name="s")

    @functools.partial(
        pl.kernel,
        out_type=jax.ShapeDtypeStruct((2 * n_nodes, 128), jnp.float32),
        mesh=mesh,
        scratch_types=[
            pltpu.VMEM((rpt, 128), jnp.int32),
            pltpu.VMEM((rpt, 128), jnp.int32),
            pltpu.VMEM((128, 128), jnp.float32),
            pltpu.VMEM((128, 128), jnp.float32),
            pltpu.VMEM((128, 128), jnp.float32),
            pltpu.SemaphoreType.DMA,
            pltpu.SemaphoreType.DMA,
            pltpu.SemaphoreType.DMA,
            pltpu.SemaphoreType.DMA,
            pltpu.VMEM_SHARED((n_nodes, 128), jnp.float32),
        ],
    )
    def k(y_hbm, src_hbm, dst_hbm, a_hbm,
          src_v, dst_v, buf0, buf1, buf2,
          gsem0, gsem1, ssem, seedsem, acc_sh):
        c = lax.axis_index("c")
        s = lax.axis_index("s")
        bufs = [buf0, buf1, buf2]
        gsems = [gsem0, gsem1]
        cbase = c * n_nodes

        seed = pltpu.async_copy(y_hbm.at[pl.ds(cbase + s * stripe, stripe)],
                                acc_sh.at[pl.ds(s * stripe, stripe)], seedsem)

        pltpu.sync_copy(src_hbm.at[pl.ds(s * rpt, rpt)], src_v)
        pltpu.sync_copy(dst_hbm.at[pl.ds(s * rpt, rpt)], dst_v)

        def off(t, _):
            sl = (t // 8, pl.ds((t % 8) * LANES, LANES))
            src_v[sl] = src_v[sl] + cbase
            return 0

        lax.fori_loop(0, rpt * 8, off, 0)

        gd = [None] * rpt
        sd = [None] * rpt
        gd[0] = pltpu.async_copy(y_hbm.at[src_v.at[0]], bufs[0], gsems[0])
        if rpt > 1:
            gd[1] = pltpu.async_copy(y_hbm.at[src_v.at[1]], bufs[1], gsems[1])
        seed.wait()
        plsc.subcore_barrier()
        for j in range(rpt):
            gd[j].wait()
            if j >= 1:
                sd[j - 1].wait()
            sd[j] = pltpu.async_copy(bufs[j % 3], acc_sh.at[dst_v.at[j]],
                                     ssem, add=True)
            if j + 2 < rpt:
                gd[j + 2] = pltpu.async_copy(
                    y_hbm.at[src_v.at[j + 2]], bufs[(j + 2) % 3],
                    gsems[j % 2])
        sd[rpt - 1].wait()
        plsc.subcore_barrier()
        pltpu.sync_copy(acc_sh.at[pl.ds(s * stripe, stripe)],
                        a_hbm.at[pl.ds(cbase + s * stripe, stripe)])

    return k(ycat, src2, dst2)


def _bn_fc_kernel(acat, degp, bc2, gamma1, beta1, fc_w, fcb1):
    n2, half = acat.shape
    n = n2 // 2
    dh = 2 * half
    d4 = 4 * dh
    blk = 512
    gblk = 256
    g_all = n // 4
    p1 = n // blk
    p2 = g_all // gblk
    nw = degp.shape[0]
    ncls = fc_w.shape[0]
    inv_n = 1.0 / float(n)

    def body(a0_ref, a1_ref, degp_ref, bc_ref, ga_ref, be_ref, fw_ref,
             fb_ref, out_ref, hg_s, st_s):
        i = pl.program_id(0)

        @pl.when(i == 0)
        def _():
            st_s[...] = jnp.zeros_like(st_s)

        @pl.when(i < p1)
        def _():
            deg = jnp.sum(degp_ref[...], axis=0) + 1.0
            dinv = lax.rsqrt(deg)[:, None]
            h0 = jnp.maximum(dinv * a0_ref[...] + bc_ref[0:1, :], 0.0)
            h1 = jnp.maximum(dinv * a1_ref[...] + bc_ref[1:2, :], 0.0)
            hcat = jnp.concatenate([h0, h1], axis=1)
            hg_s[pl.ds(i * (blk // 4), blk // 4), :] = hcat.reshape(
                blk // 4, d4)
            row_s = jnp.concatenate(
                [jnp.sum(h0, axis=0), jnp.sum(h1, axis=0)])
            row_q = jnp.concatenate(
                [jnp.sum(h0 * h0, axis=0), jnp.sum(h1 * h1, axis=0)])
            st_s[0:1, :] += row_s[None, :]
            st_s[1:2, :] += row_q[None, :]

        @pl.when(i >= p1)
        def _():
            j = i - p1
            mean = st_s[0:1, :] * inv_n
            ex2 = st_s[1:2, :] * inv_n
            var = ex2 - mean * mean
            rstd = lax.rsqrt(var + 1e-5)
            scale = ga_ref[0:1, :] * rstd
            off = be_ref[0:1, :] - mean * scale
            scale4 = jnp.concatenate([scale] * 4, axis=1)
            off4 = jnp.concatenate([off] * 4, axis=1)
            hgn = hg_s[pl.ds(j * gblk, gblk), :] * scale4 + off4
            ltT = lax.dot_general(
                fw_ref[...], hgn, (((1,), (1,)), ((), ())),
                preferred_element_type=jnp.float32) + fb_ref[...]
            m = jnp.max(ltT, axis=0, keepdims=True)
            lse = m + jnp.log(
                jnp.sum(jnp.exp(ltT - m), axis=0, keepdims=True))
            lsT = ltT - lse
            gsrc = lax.broadcasted_iota(jnp.int32, (gblk, 4 * gblk), 0)
            gdst = lax.broadcasted_iota(jnp.int32, (gblk, 4 * gblk), 1) // 4
            rep = (gsrc == gdst).astype(jnp.float32)
            out_ref[...] = lax.dot_general(
                lsT, rep, (((1,), (0,)), ((), ())),
                preferred_element_type=jnp.float32)

    return pl.pallas_call(
        body,
        grid=(p1 + p2,),
        in_specs=[
            pl.BlockSpec((blk, half), lambda i: (jnp.minimum(i, p1 - 1), 0)),
            pl.BlockSpec((blk, half),
                         lambda i: (p1 + jnp.minimum(i, p1 - 1), 0)),
            pl.BlockSpec((nw, blk), lambda i: (0, jnp.minimum(i, p1 - 1))),
            pl.BlockSpec((2, half), lambda i: (0, 0)),
            pl.BlockSpec((1, dh), lambda i: (0, 0)),
            pl.BlockSpec((1, dh), lambda i: (0, 0)),
            pl.BlockSpec((ncls, d4), lambda i: (0, 0)),
            pl.BlockSpec((ncls, 1), lambda i: (0, 0)),
        ],
        out_specs=pl.BlockSpec(
            (ncls, 4 * gblk), lambda i: (0, jnp.maximum(i - p1, 0))),
        out_shape=jax.ShapeDtypeStruct((ncls, n), jnp.float32),
        scratch_shapes=[
            pltpu.VMEM((g_all, d4), jnp.float32),
            pltpu.VMEM((8, dh), jnp.float32),
        ],
    )(acat, acat, degp, bc2, gamma1, beta1, fc_w, fcb1)


def kernel(x, edge_index, num_graphs, W, b_conv, gamma, beta, fc_W, fc_b):
    del num_graphs
    n, _ = x.shape
    dh = W.shape[1]
    src = edge_index[0]
    dst = edge_index[1]
    e = src.shape[0]

    degp = _deg_kernel(dst, n)
    ycat = _matmul_kernel(x, W, degp).reshape(2 * n, dh // 2)
    src2 = src.reshape(e // 128, 128)
    dst2 = dst.reshape(e // 128, 128)
    acat = _scatter_kernel(ycat, src2, dst2, n)
    bc2 = b_conv.reshape(2, dh // 2)
    outT = _bn_fc_kernel(acat, degp, bc2, gamma.reshape(1, dh),
                         beta.reshape(1, dh), fc_W, fc_b.reshape(-1, 1))
    return outT.T
